# baseline (speedup 1.0000x reference)
import jax
import jax.numpy as jnp
from jax.experimental import pallas as pl
from jax.experimental.pallas import tpu as pltpu


def _body(alpha_ref, beta_ref, lc_ref, g_ref, label_ref, out_ref):
    a = alpha_ref[0]
    b = beta_ref[0]
    c = lc_ref[0]
    g = g_ref[...]
    out_ref[...] = a + b * g + c * (label_ref[...] * g)


def kernel(user, item, g_s, label, alpha, beta, label_coef):
    B = g_s.shape[0]
    g2 = g_s.reshape(8, B // 8)
    l2 = label.reshape(8, B // 8)
    out = pl.pallas_call(
        _body,
        out_shape=jax.ShapeDtypeStruct(g2.shape, jnp.float32),
        in_specs=[
            pl.BlockSpec(memory_space=pltpu.SMEM),
            pl.BlockSpec(memory_space=pltpu.SMEM),
            pl.BlockSpec(memory_space=pltpu.SMEM),
            pl.BlockSpec(memory_space=pltpu.VMEM),
            pl.BlockSpec(memory_space=pltpu.VMEM),
        ],
        out_specs=pl.BlockSpec(memory_space=pltpu.VMEM),
    )(alpha, beta, label_coef, g2, l2)
    return out.reshape(B)
